# Initial kernel scaffold; baseline (speedup 1.0000x reference)
import jax
import jax.numpy as jnp
from jax.experimental import pallas as pl


def kernel(p, x, o, params):
    raise NotImplementedError("write your pallas kernel here")


# speedup vs baseline: 13.1451x; 13.1451x over previous
import functools

import jax
import jax.numpy as jnp
from jax import lax
from jax.experimental import pallas as pl
from jax.experimental.pallas import tpu as pltpu
from jax.experimental.pallas import tpu_sc as plsc

N = 8192
D_IN = 64
D_OUT = 128
H = D_OUT // 2
K = 16
STRIDE = 4
EPS = 1e-5
COUNT = N * STRIDE // (STRIDE + 1)
CPAD = 6656

_SC_CORES = 2
_SC_SUBCORES = 16
_SC_WORKERS = _SC_CORES * _SC_SUBCORES


@functools.lru_cache(maxsize=None)
def _make_sc_gather(V, D, B):
    del V
    b_per_w = B // _SC_WORKERS
    chunk = min(b_per_w, 512)
    n_chunks = b_per_w // chunk
    mesh = plsc.VectorSubcoreMesh(core_axis_name="c", subcore_axis_name="s")

    def body(table_hbm, idx_hbm, out_hbm, idx_v, rows_v, sem):
        wid = lax.axis_index("s") * _SC_CORES + lax.axis_index("c")
        for c in range(n_chunks):
            base = wid * b_per_w + c * chunk
            pltpu.sync_copy(idx_hbm.at[pl.ds(base, chunk)], idx_v)
            pltpu.async_copy(table_hbm.at[idx_v], rows_v, sem).wait()
            pltpu.sync_copy(rows_v, out_hbm.at[pl.ds(base, chunk)])

    return pl.kernel(
        body,
        mesh=mesh,
        out_type=jax.ShapeDtypeStruct((B, D), jnp.float32),
        scratch_types=[
            pltpu.VMEM((chunk,), jnp.int32),
            pltpu.VMEM((chunk, D), jnp.float32),
            pltpu.SemaphoreType.DMA,
        ],
        compiler_params=pltpu.CompilerParams(use_tc_tiling_on_sc=False),
    )


def _sc_gather(table, idx_flat):
    B = idx_flat.shape[0]
    V, D = table.shape
    return _make_sc_gather(V, D, B)(table, idx_flat)


def _mlp0_body(x_ref, wt_ref, b_ref, g_ref, be_ref, o_ref):
    z = jnp.dot(x_ref[...], wt_ref[...], preferred_element_type=jnp.float32)
    z = z + b_ref[...]
    n = z.shape[0]
    s = jnp.sum(z, axis=0, keepdims=True)
    q = jnp.sum(z * z, axis=0, keepdims=True)
    m = s / n
    v = q / n - m * m
    o_ref[...] = jnp.maximum((z - m) / jnp.sqrt(v + EPS) * g_ref[...] + be_ref[...], 0.0)


def _mlp0(x, wt, b, g, be):
    return pl.pallas_call(
        _mlp0_body,
        out_shape=jax.ShapeDtypeStruct((N, D_IN), jnp.float32),
    )(x, wt, b, g, be)


_KNN_BLK = 256


def _knn_body(pb_ref, pt_ref, idx_ref, d_ref):
    pb = pb_ref[...]
    pt = pt_ref[...]
    sq_b = jnp.sum(pb * pb, axis=1, keepdims=True)
    sq_a = jnp.sum(pt * pt, axis=0, keepdims=True)
    d = sq_b + sq_a - 2.0 * jnp.dot(pb, pt, preferred_element_type=jnp.float32)
    d_ref[...] = d
    col = lax.broadcasted_iota(jnp.int32, (_KNN_BLK, N), 1)
    cols = []
    for _ in range(K):
        d = d_ref[...]
        m = jnp.min(d, axis=1, keepdims=True)
        j = jnp.min(jnp.where(d == m, col, N), axis=1, keepdims=True)
        cols.append(j)
        d_ref[...] = jnp.where(col == j, jnp.float32(jnp.inf), d)
    idx_ref[...] = jnp.concatenate(cols, axis=1)


def _knn(p16, p16t):
    nb = N // _KNN_BLK
    return pl.pallas_call(
        _knn_body,
        grid=(nb,),
        in_specs=[
            pl.BlockSpec((_KNN_BLK, 16), lambda i: (i, 0)),
            pl.BlockSpec((16, N), lambda i: (0, 0)),
        ],
        out_specs=pl.BlockSpec((_KNN_BLK, K), lambda i: (i, 0)),
        out_shape=jax.ShapeDtypeStruct((N, K), jnp.int32),
        scratch_shapes=[pltpu.VMEM((_KNN_BLK, N), jnp.float32)],
    )(p16, p16t)


_RFE_BLK = 512
_RFE_NB = N // _RFE_BLK
_NL = N * K


def _rfe_t1(pg_ref, p_ref, a_ref, b_ref, c_ref, c1b_ref):
    rows = _RFE_BLK * K
    pgb = pg_ref[...]
    ctr = jnp.broadcast_to(
        p_ref[...].reshape(_RFE_BLK, 1, 16), (_RFE_BLK, K, 16)).reshape(rows, 16)
    pg0 = jnp.broadcast_to(
        pg_ref[...].reshape(_RFE_BLK, K, 16)[:, 0:1, :],
        (_RFE_BLK, K, 16)).reshape(rows, 16)
    gx = pgb - ctr
    gx0 = pg0 - ctr
    diff = pg0 - pgb
    dist = jnp.sqrt(jnp.sum(diff * diff, axis=1, keepdims=True) + 1e-12)
    return (jnp.dot(gx0, a_ref[...], preferred_element_type=jnp.float32)
            + jnp.dot(gx, b_ref[...], preferred_element_type=jnp.float32)
            + dist * c_ref[...] + c1b_ref[...])


def _rfe_body(pg_ref, p_ref, gf_ref,
              a_ref, b_ref, c_ref, c1b_ref, bn1g_ref, bn1b_ref,
              c2w_ref, c2b_ref, sw_ref, mw_ref, mb_ref,
              bng_ref, bnb_ref,
              out_ref,
              zbuf, s1, q1, sz, qz):
    ph = pl.program_id(0)
    i = pl.program_id(1)

    @pl.when((ph == 0) & (i == 0))
    def _():
        s1[...] = jnp.zeros_like(s1)
        q1[...] = jnp.zeros_like(q1)

    @pl.when(ph == 0)
    def _():
        t1 = _rfe_t1(pg_ref, p_ref, a_ref, b_ref, c_ref, c1b_ref)
        s1[...] += jnp.sum(t1, axis=0, keepdims=True)
        q1[...] += jnp.sum(t1 * t1, axis=0, keepdims=True)

    @pl.when((ph == 1) & (i == 0))
    def _():
        sz[...] = jnp.zeros_like(sz)
        qz[...] = jnp.zeros_like(qz)

    @pl.when(ph == 1)
    def _():
        m1 = s1[...] / _NL
        v1 = q1[...] / _NL - m1 * m1
        sc1 = bn1g_ref[...] / jnp.sqrt(v1 + EPS)
        sh1 = bn1b_ref[...] - m1 * sc1
        t1 = _rfe_t1(pg_ref, p_ref, a_ref, b_ref, c_ref, c1b_ref)
        t = jnp.maximum(t1 * sc1 + sh1, 0.0)
        pc = jnp.dot(t, c2w_ref[...], preferred_element_type=jnp.float32) + c2b_ref[...]
        px = jnp.concatenate([pc, gf_ref[...]], axis=-1)
        sc = jnp.dot(px, sw_ref[...], preferred_element_type=jnp.float32)
        s3 = sc.reshape(_RFE_BLK, K, D_OUT)
        mx = jnp.max(s3, axis=1, keepdims=True)
        e = jnp.exp(s3 - mx)
        den = jnp.sum(e, axis=1, keepdims=True)
        scores = e / den
        feats = jnp.sum(scores * px.reshape(_RFE_BLK, K, D_OUT), axis=1)
        z = jnp.dot(feats, mw_ref[...], preferred_element_type=jnp.float32) + mb_ref[...]
        zbuf[pl.ds(i * _RFE_BLK, _RFE_BLK), :] = z
        sz[...] += jnp.sum(z, axis=0, keepdims=True)
        qz[...] += jnp.sum(z * z, axis=0, keepdims=True)

    @pl.when(ph == 2)
    def _():
        mzv = sz[...] / N
        vzv = qz[...] / N - mzv * mzv
        scz = bng_ref[...] / jnp.sqrt(vzv + EPS)
        shz = bnb_ref[...] - mzv * scz
        z = zbuf[pl.ds(i * _RFE_BLK, _RFE_BLK), :]
        out_ref[...] = jnp.maximum(z * scz + shz, 0.0)


def _rfe(pg_flat, p16, gf_flat, pr):
    w1t = pr['c1W'].T
    a = jnp.pad(w1t[0:3] + w1t[6:9], ((0, 13), (0, 0)))
    b = jnp.pad(w1t[3:6] - w1t[6:9], ((0, 13), (0, 0)))
    c = w1t[9:10]
    wargs = (a, b, c, pr['c1b'].reshape(1, -1),
             pr['bn1g'].reshape(1, -1), pr['bn1b'].reshape(1, -1),
             pr['c2W'].T, pr['c2b'].reshape(1, -1),
             pr['sW'].T, pr['mW'].T, pr['mb'].reshape(1, -1),
             pr['bng'].reshape(1, -1), pr['bnb'].reshape(1, -1))
    rows = _RFE_BLK * K
    full = lambda shape: pl.BlockSpec(shape, lambda ph, i: tuple(0 for _ in shape))
    return pl.pallas_call(
        _rfe_body,
        grid=(3, _RFE_NB),
        in_specs=[
            pl.BlockSpec((rows, 16), lambda ph, i: (i, 0)),
            pl.BlockSpec((_RFE_BLK, 16), lambda ph, i: (i, 0)),
            pl.BlockSpec((rows, D_IN), lambda ph, i: (i, 0)),
            full((16, D_OUT)), full((16, D_OUT)), full((1, D_OUT)), full((1, D_OUT)),
            full((1, D_OUT)), full((1, D_OUT)),
            full((D_OUT, H)), full((1, H)), full((D_OUT, D_OUT)),
            full((D_OUT, H)), full((1, H)), full((1, H)), full((1, H)),
        ],
        out_specs=pl.BlockSpec((_RFE_BLK, D_IN), lambda ph, i: (i, 0)),
        out_shape=jax.ShapeDtypeStruct((N, D_IN), jnp.float32),
        scratch_shapes=[
            pltpu.VMEM((N, D_IN), jnp.float32),
            pltpu.VMEM((1, D_OUT), jnp.float32),
            pltpu.VMEM((1, D_OUT), jnp.float32),
            pltpu.VMEM((1, H), jnp.float32),
            pltpu.VMEM((1, H), jnp.float32),
        ],
    )(pg_flat, p16, gf_flat, *wargs)


def _final_body(x_ref, x2_ref, x3_ref,
                w01_ref, b01_ref, g01_ref, be01_ref,
                w1_ref, b1_ref, g1_ref, be1_ref, o_ref):
    def bnrelu(z, g, be):
        n = z.shape[0]
        s = jnp.sum(z, axis=0, keepdims=True)
        q = jnp.sum(z * z, axis=0, keepdims=True)
        m = s / n
        v = q / n - m * m
        return jnp.maximum((z - m) / jnp.sqrt(v + EPS) * g + be, 0.0)

    za = jnp.dot(x_ref[...], w01_ref[...], preferred_element_type=jnp.float32) + b01_ref[...]
    w1 = w1_ref[...]
    zb = (jnp.dot(x2_ref[...], w1[:D_IN, :], preferred_element_type=jnp.float32)
          + jnp.dot(x3_ref[...], w1[D_IN:, :], preferred_element_type=jnp.float32)
          + b1_ref[...])
    o_ref[...] = (bnrelu(za, g01_ref[...], be01_ref[...])
                  + bnrelu(zb, g1_ref[...], be1_ref[...]))


def _final(x, x2, x3, m01, m1):
    return pl.pallas_call(
        _final_body,
        out_shape=jax.ShapeDtypeStruct((N, D_OUT), jnp.float32),
    )(x, x2, x3,
      m01['W'].T, m01['b'].reshape(1, -1), m01['g'].reshape(1, -1), m01['be'].reshape(1, -1),
      m1['W'].T, m1['b'].reshape(1, -1), m1['g'].reshape(1, -1), m1['be'].reshape(1, -1))


def _fps_body(px_ref, py_ref, pz_ref, o_ref):
    px = px_ref[...]
    py = py_ref[...]
    pz = pz_ref[...]
    fio = (lax.broadcasted_iota(jnp.int32, (64, 128), 0) * 128
           + lax.broadcasted_iota(jnp.int32, (64, 128), 1))
    fio52 = (lax.broadcasted_iota(jnp.int32, (52, 128), 0) * 128
             + lax.broadcasted_iota(jnp.int32, (52, 128), 1))
    zf = jnp.zeros((64, 128), jnp.float32)

    def body(j, carry):
        dists, fmat, last = carry
        msk = fio == last
        lx = jnp.sum(jnp.where(msk, px, zf))
        ly = jnp.sum(jnp.where(msk, py, zf))
        lz = jnp.sum(jnp.where(msk, pz, zf))
        dx = px - lx
        dy = py - ly
        dz = pz - lz
        d = (dx * dx + dz * dz) + dy * dy
        dists = jnp.minimum(dists, d)
        mx = jnp.max(dists)
        nxt = jnp.min(jnp.where(dists == mx, fio, N))
        fmat = jnp.where(fio52 == j, nxt, fmat)
        return dists, fmat, nxt

    dists0 = jnp.full((64, 128), 1e10, jnp.float32)
    fmat0 = jnp.zeros((52, 128), jnp.int32)
    _, fmat, _ = lax.fori_loop(1, COUNT, body, (dists0, fmat0, jnp.int32(0)))
    o_ref[...] = fmat


def _fps(px, py, pz):
    return pl.pallas_call(
        _fps_body,
        out_shape=jax.ShapeDtypeStruct((52, 128), jnp.int32),
    )(px, py, pz)


def kernel(p, x, o, params):
    p16 = jnp.pad(p, ((0, 0), (0, 13)))
    p16t = p16.T

    m0 = params['mlp0']
    x1 = _mlp0(x, m0['W'].T, m0['b'].reshape(1, -1),
               m0['g'].reshape(1, -1), m0['be'].reshape(1, -1))

    idx = _knn(p16, p16t)
    idx_flat = idx.reshape(N * K)

    pg_flat = _sc_gather(p16, idx_flat)
    gf1 = _sc_gather(x1, idx_flat)
    x2 = _rfe(pg_flat, p16, gf1, params['rfe1'])
    gf2 = _sc_gather(x2, idx_flat)
    x3 = _rfe(pg_flat, p16, gf2, params['rfe2'])

    out = _final(x, x2, x3, params['mlp01'], params['mlp1'])

    fmat = _fps(p[:, 0].reshape(64, 128), p[:, 1].reshape(64, 128),
                p[:, 2].reshape(64, 128))
    fidx = fmat.reshape(CPAD)

    n_p = _sc_gather(p16, fidx)[:COUNT, :3]
    n_x = _sc_gather(out, fidx)[:COUNT, :]
    n_o = (o * STRIDE // (STRIDE + 1)).astype(jnp.int32)
    return n_p, n_x, n_o
